# baseline (speedup 1.0000x reference)
import functools

import jax
import jax.numpy as jnp
from jax import lax
from jax.experimental import pallas as pl
from jax.experimental.pallas import tpu as pltpu
from jax.experimental.pallas import tpu_sc as plsc

N = 262144
EPS = 1e-7
NW = 32
PA = N // NW
ITERS = PA // 16

LN2 = 0.6931471805599453
SQRT2 = 1.4142135623730951
_LOG_COEF = (
    6.4325946848757e-08,
    1.0000040903431004,
    -0.5000199313315633,
    0.3329959690927211,
    -0.24886373808989276,
    0.2065534306267291,
    -0.18852481818682676,
    0.11589596284372891,
)


def _ln(q):
    bits = lax.bitcast_convert_type(q, jnp.int32)
    e = (bits >> 23) - 127
    m = lax.bitcast_convert_type((bits & 0x007FFFFF) | 0x3F800000, jnp.float32)
    big = m > SQRT2
    m = jnp.where(big, m * 0.5, m)
    ef = e.astype(jnp.float32) + jnp.where(big, 1.0, 0.0)
    u = m - 1.0
    p = jnp.full_like(q, _LOG_COEF[7])
    for c in _LOG_COEF[6::-1]:
        p = p * u + c
    return p + ef * LN2


def _sc_cls(ts_hbm, os_hbm, out_hbm, ts_v, os_v, acc_v, s0, s1):
    wid = lax.axis_index("s") * 2 + lax.axis_index("c")
    abase = wid * PA

    c0 = pltpu.async_copy(ts_hbm.at[pl.ds(abase, PA)], ts_v, s0)
    c1 = pltpu.async_copy(os_hbm.at[pl.ds(abase, PA)], os_v, s1)
    c0.wait()
    c1.wait()

    def one(a0):
        t = ts_v[pl.ds(a0, 16)]
        p = os_v[pl.ds(a0, 16)]
        valid = t >= 0.0
        pos = t > 0.0
        p = jnp.minimum(jnp.maximum(p, EPS), 1.0 - EPS)
        q = jnp.where(pos, p, 1.0 - p)
        bce = -_ln(q)
        return (jnp.where(valid, bce, 0.0),
                jnp.where(valid, 1.0, 0.0),
                jnp.where(pos, 1.0, 0.0))

    def body(i, carry):
        acc_bce, acc_nv, acc_np = carry
        a0 = pl.multiple_of(i * 32, 32)
        b0, v0, p0 = one(a0)
        b1, v1, p1 = one(a0 + 16)
        return (acc_bce + (b0 + b1), acc_nv + (v0 + v1), acc_np + (p0 + p1))

    z = jnp.zeros((16,), jnp.float32)
    acc_bce, acc_nv, acc_np = lax.fori_loop(0, ITERS // 2, body, (z, z, z))

    acc_v[pl.ds(0, 16)] = acc_bce
    acc_v[pl.ds(16, 16)] = acc_nv
    acc_v[pl.ds(32, 16)] = acc_np
    pltpu.sync_copy(acc_v, out_hbm.at[wid])


_sc_call = functools.partial(
    pl.kernel,
    out_type=jax.ShapeDtypeStruct((NW, 48), jnp.float32),
    mesh=plsc.VectorSubcoreMesh(core_axis_name="c", subcore_axis_name="s"),
    scratch_types=[
        pltpu.VMEM((PA,), jnp.float32),
        pltpu.VMEM((PA,), jnp.float32),
        pltpu.VMEM((48,), jnp.float32),
        pltpu.SemaphoreType.DMA,
        pltpu.SemaphoreType.DMA,
    ],
    compiler_params=pltpu.CompilerParams(needs_layout_passes=False),
)(_sc_cls)


_GRID = 8
_RB = (N // 128) // _GRID
_DRB = 4 * _RB


def _tc_reg_body(ts_ref, td_ref, od_ref, out_ref):
    i = pl.program_id(0)
    d = od_ref[...] - td_ref[...]
    ad = jnp.abs(d)
    m = jnp.minimum(ad, 1.0)
    f = m * (ad - 0.5 * m)
    g = jnp.sum(f.reshape(_RB, 4, 128), axis=1)
    pos = (ts_ref[...] > 0.0).astype(jnp.float32)
    blk = jnp.sum(pos * g)

    @pl.when(i == 0)
    def _init():
        out_ref[0, 0] = blk

    @pl.when(i > 0)
    def _acc():
        out_ref[0, 0] += blk


def _combine_body(sc_ref, reg_ref, o_ref):
    x = sc_ref[...]
    bce = jnp.sum(x[:, 0:16])
    nv = jnp.sum(x[:, 16:32])
    npos = jnp.sum(x[:, 32:48])
    reg = reg_ref[0, 0]
    o_ref[0, 0] = bce / jnp.maximum(nv, 1.0) + 10.0 * reg / jnp.maximum(EPS, npos)


def kernel(target_deltas, target_scores, output_deltas, output_scores):
    ts = target_scores.reshape(N)
    osc = output_scores.reshape(N)
    ts2 = target_scores.reshape(N // 128, 128)
    td8 = target_deltas.reshape(N // 128, 128, 4).transpose(0, 2, 1).reshape(N // 32, 128)
    od8 = output_deltas.reshape(N // 128, 128, 4).transpose(0, 2, 1).reshape(N // 32, 128)

    reg_sum = pl.pallas_call(
        _tc_reg_body,
        grid=(_GRID,),
        in_specs=[
            pl.BlockSpec((_RB, 128), lambda i: (i, 0)),
            pl.BlockSpec((_DRB, 128), lambda i: (i, 0)),
            pl.BlockSpec((_DRB, 128), lambda i: (i, 0)),
        ],
        out_specs=pl.BlockSpec(
            (1, 1), lambda i: (0, 0), memory_space=pltpu.SMEM),
        out_shape=jax.ShapeDtypeStruct((1, 1), jnp.float32),
    )(ts2, td8, od8)

    sc_partials = _sc_call(ts, osc)

    out = pl.pallas_call(
        _combine_body,
        out_shape=jax.ShapeDtypeStruct((1, 1), jnp.float32),
        in_specs=[
            pl.BlockSpec(memory_space=pltpu.VMEM),
            pl.BlockSpec(memory_space=pltpu.SMEM),
        ],
        out_specs=pl.BlockSpec(memory_space=pltpu.SMEM),
    )(sc_partials, reg_sum)
    return out[0, 0]


# speedup vs baseline: 1.0081x; 1.0081x over previous
import functools

import jax
import jax.numpy as jnp
from jax import lax
from jax.experimental import pallas as pl
from jax.experimental.pallas import tpu as pltpu
from jax.experimental.pallas import tpu_sc as plsc

N = 262144
EPS = 1e-7
NW = 32
PA = N // NW
ITERS = PA // 16

LN2 = 0.6931471805599453
SQRT2 = 1.4142135623730951
_LOG_COEF = (
    6.4325946848757e-08,
    1.0000040903431004,
    -0.5000199313315633,
    0.3329959690927211,
    -0.24886373808989276,
    0.2065534306267291,
    -0.18852481818682676,
    0.11589596284372891,
)


def _ln(q):
    bits = lax.bitcast_convert_type(q, jnp.int32)
    e = (bits >> 23) - 127
    m = lax.bitcast_convert_type((bits & 0x007FFFFF) | 0x3F800000, jnp.float32)
    big = m > SQRT2
    m = jnp.where(big, m * 0.5, m)
    ef = e.astype(jnp.float32) + jnp.where(big, 1.0, 0.0)
    u = m - 1.0
    p = jnp.full_like(q, _LOG_COEF[7])
    for c in _LOG_COEF[6::-1]:
        p = p * u + c
    return p + ef * LN2


def _sc_cls(ts_hbm, os_hbm, out_hbm, ts_v, os_v, acc_v, s0, s1):
    wid = lax.axis_index("s") * 2 + lax.axis_index("c")
    abase = wid * PA

    c0 = pltpu.async_copy(ts_hbm.at[pl.ds(abase, PA)], ts_v, s0)
    c1 = pltpu.async_copy(os_hbm.at[pl.ds(abase, PA)], os_v, s1)
    c0.wait()
    c1.wait()

    def one(a0):
        t = ts_v[pl.ds(a0, 16)]
        p = os_v[pl.ds(a0, 16)]
        valid = t >= 0.0
        pos = t > 0.0
        p = jnp.minimum(jnp.maximum(p, EPS), 1.0 - EPS)
        q = jnp.where(pos, p, 1.0 - p)
        bce = -_ln(q)
        return (jnp.where(valid, bce, 0.0),
                jnp.where(valid, 1.0, 0.0),
                jnp.where(pos, 1.0, 0.0))

    def body(i, carry):
        acc_bce, acc_nv, acc_np = carry
        a0 = pl.multiple_of(i * 32, 32)
        b0, v0, p0 = one(a0)
        b1, v1, p1 = one(a0 + 16)
        return (acc_bce + (b0 + b1), acc_nv + (v0 + v1), acc_np + (p0 + p1))

    z = jnp.zeros((16,), jnp.float32)
    acc_bce, acc_nv, acc_np = lax.fori_loop(0, ITERS // 2, body, (z, z, z))

    acc_v[pl.ds(0, 16)] = acc_bce
    acc_v[pl.ds(16, 16)] = acc_nv
    acc_v[pl.ds(32, 16)] = acc_np
    pltpu.sync_copy(acc_v, out_hbm.at[wid])


_sc_call = functools.partial(
    pl.kernel,
    out_type=jax.ShapeDtypeStruct((NW, 48), jnp.float32),
    mesh=plsc.VectorSubcoreMesh(core_axis_name="c", subcore_axis_name="s"),
    scratch_types=[
        pltpu.VMEM((PA,), jnp.float32),
        pltpu.VMEM((PA,), jnp.float32),
        pltpu.VMEM((48,), jnp.float32),
        pltpu.SemaphoreType.DMA,
        pltpu.SemaphoreType.DMA,
    ],
    compiler_params=pltpu.CompilerParams(needs_layout_passes=False),
)(_sc_cls)


_GRID = 8
_RB = (N // 128) // _GRID
_DRB = 4 * _RB


def _tc_reg_body(ts_hbm, td_hbm, od_hbm, out_ref, ts_b, td_b, od_b, sem):
    def start(k, slot):
        pltpu.make_async_copy(
            ts_hbm.at[pl.ds(k * _RB, _RB), :], ts_b.at[slot], sem.at[slot, 0]
        ).start()
        pltpu.make_async_copy(
            td_hbm.at[pl.ds(k * _DRB, _DRB), :], td_b.at[slot], sem.at[slot, 1]
        ).start()
        pltpu.make_async_copy(
            od_hbm.at[pl.ds(k * _DRB, _DRB), :], od_b.at[slot], sem.at[slot, 2]
        ).start()

    def wait(k, slot):
        pltpu.make_async_copy(
            ts_hbm.at[pl.ds(k * _RB, _RB), :], ts_b.at[slot], sem.at[slot, 0]
        ).wait()
        pltpu.make_async_copy(
            td_hbm.at[pl.ds(k * _DRB, _DRB), :], td_b.at[slot], sem.at[slot, 1]
        ).wait()
        pltpu.make_async_copy(
            od_hbm.at[pl.ds(k * _DRB, _DRB), :], od_b.at[slot], sem.at[slot, 2]
        ).wait()

    start(0, 0)

    def body(k, acc):
        slot = lax.rem(k, 2)
        nslot = lax.rem(k + 1, 2)

        @pl.when(k + 1 < _GRID)
        def _pf():
            start(k + 1, nslot)

        wait(k, slot)
        d = od_b[slot] - td_b[slot]
        ad = jnp.abs(d)
        m = jnp.minimum(ad, 1.0)
        f = m * (ad - 0.5 * m)
        g = jnp.sum(f.reshape(_RB, 4, 128), axis=1)
        pos = (ts_b[slot] > 0.0).astype(jnp.float32)
        return acc + jnp.sum(pos * g)

    out_ref[0, 0] = lax.fori_loop(0, _GRID, body, 0.0)


def _combine_body(sc_ref, reg_ref, o_ref):
    x = sc_ref[...]
    bce = jnp.sum(x[:, 0:16])
    nv = jnp.sum(x[:, 16:32])
    npos = jnp.sum(x[:, 32:48])
    reg = reg_ref[0, 0]
    o_ref[0, 0] = bce / jnp.maximum(nv, 1.0) + 10.0 * reg / jnp.maximum(EPS, npos)


def kernel(target_deltas, target_scores, output_deltas, output_scores):
    ts = target_scores.reshape(N)
    osc = output_scores.reshape(N)
    ts2 = target_scores.reshape(N // 128, 128)
    td8 = target_deltas.reshape(N // 128, 128, 4).transpose(0, 2, 1).reshape(N // 32, 128)
    od8 = output_deltas.reshape(N // 128, 128, 4).transpose(0, 2, 1).reshape(N // 32, 128)

    reg_sum = pl.pallas_call(
        _tc_reg_body,
        in_specs=[
            pl.BlockSpec(memory_space=pltpu.MemorySpace.HBM),
            pl.BlockSpec(memory_space=pltpu.MemorySpace.HBM),
            pl.BlockSpec(memory_space=pltpu.MemorySpace.HBM),
        ],
        out_specs=pl.BlockSpec(memory_space=pltpu.SMEM),
        out_shape=jax.ShapeDtypeStruct((1, 1), jnp.float32),
        scratch_shapes=[
            pltpu.VMEM((2, _RB, 128), jnp.float32),
            pltpu.VMEM((2, _DRB, 128), jnp.float32),
            pltpu.VMEM((2, _DRB, 128), jnp.float32),
            pltpu.SemaphoreType.DMA((2, 3)),
        ],
    )(ts2, td8, od8)

    sc_partials = _sc_call(ts, osc)

    out = pl.pallas_call(
        _combine_body,
        out_shape=jax.ShapeDtypeStruct((1, 1), jnp.float32),
        in_specs=[
            pl.BlockSpec(memory_space=pltpu.VMEM),
            pl.BlockSpec(memory_space=pltpu.SMEM),
        ],
        out_specs=pl.BlockSpec(memory_space=pltpu.SMEM),
    )(sc_partials, reg_sum)
    return out[0, 0]
